# baseline (speedup 1.0000x reference)
import jax
import jax.numpy as jnp
from jax import lax
from jax.experimental import pallas as pl
from jax.experimental.pallas import tpu as pltpu
from jax.experimental.pallas import tpu_sc as plsc

_K = 10
_N = 8192
_E = 50000


_R = 512


def _oem_sort_network(n):
    comps = []

    def oddeven_merge(lo, n2, r):
        step = r * 2
        if step < n2:
            oddeven_merge(lo, n2, step)
            oddeven_merge(lo + r, n2, step)
            for i in range(lo + r, lo + n2 - r, step):
                comps.append((i, i + r))
        else:
            comps.append((lo, lo + r))

    def sort_range(lo, hi):
        if hi - lo >= 1:
            mid = lo + (hi - lo) // 2
            sort_range(lo, mid)
            sort_range(mid + 1, hi)
            oddeven_merge(lo, hi - lo + 1, 1)

    sort_range(0, n - 1)
    return comps


def _pruned_topk_network(n, k):
    needed = set(range(k))
    ops = []
    for (i, j) in reversed(_oem_sort_network(n)):
        ni, nj = i in needed, j in needed
        if not (ni or nj):
            continue
        ops.append((i, j, ni, nj))
        needed.add(i)
        needed.add(j)
    return ops[::-1]


_NET = _pruned_topk_network(_N // 128, _K + 1)


def _dtm_body(pts_blk_ref, ptsT_ref, out_ref):
    bt = ptsT_ref[...]
    a = pts_blk_ref[...]
    sqr = jnp.sum(a * a, axis=1, keepdims=True)
    sqc = jnp.sum(bt * bt, axis=0, keepdims=True)
    G = jnp.dot(
        a.astype(jnp.bfloat16),
        bt.astype(jnp.bfloat16),
        preferred_element_type=jnp.float32,
    )
    Gp = sqc - 2.0 * G
    R = a.shape[0]
    INF = jnp.float32(jnp.inf)

    def insert(lst, x):
        mx = [jnp.maximum(lst[i], x) for i in range(len(lst) - 1)]
        lst[0] = jnp.minimum(lst[0], x)
        for i in range(1, len(lst)):
            lst[i] = jnp.minimum(lst[i], mx[i - 1])

    s = [jnp.full((R, 128), INF, jnp.float32) for _ in range(_K + 1)]
    h = [jnp.full((R, 128), INF, jnp.float32) for _ in range(5)]
    q = [jnp.full((R, 128), INF, jnp.float32) for _ in range(3)]
    r = [jnp.full((R, 128), INF, jnp.float32) for _ in range(2)]
    for v in range(0, _N // 128, 4):
        d = [sqr + Gp[:, (v + u) * 128:(v + u + 1) * 128] for u in range(4)]
        for (i, j) in ((0, 1), (2, 3), (0, 2), (1, 3), (1, 2)):
            lo, hi = jnp.minimum(d[i], d[j]), jnp.maximum(d[i], d[j])
            d[i], d[j] = lo, hi
        insert(s, d[0])
        insert(h, d[1])
        insert(q, d[2])
        insert(r, d[3])
    for x in h + q + r:
        insert(s, x)
    lane = lax.broadcasted_iota(jnp.int32, (R, 128), 1)
    acc = jnp.zeros((R, 1), jnp.float32)
    for t in range(_K + 1):
        m = jnp.min(s[0], axis=1, keepdims=True)
        if t > 0:
            acc = acc + jnp.maximum(m, 1e-12)
        if t < _K:
            hit = s[0] == m
            li = jnp.min(
                jnp.where(hit, lane, jnp.int32(999)), axis=1, keepdims=True
            )
            hf = lane == li
            for i in range(_K):
                s[i] = jnp.where(hf, s[i + 1], s[i])
            s[_K] = jnp.where(hf, INF, s[_K])
    out_ref[...] = jnp.sqrt(acc * (1.0 / _K))


def _dtm(pts, ptsT):
    out = pl.pallas_call(
        _dtm_body,
        grid=(_N // _R,),
        in_specs=[
            pl.BlockSpec((_R, 3), lambda i: (i, 0)),
            pl.BlockSpec((3, _N), lambda i: (0, 0)),
        ],
        out_specs=pl.BlockSpec((_R, 1), lambda i: (i, 0)),
        out_shape=jax.ShapeDtypeStruct((_N, 1), jnp.float32),
    )(pts, ptsT)
    return out[:, 0]


_NW = 32
_EPW = 1568
_EPAD = _NW * _EPW


def _edge_body(ptsT_hbm, dtm_hbm, eu_hbm, ev_hbm, out_hbm, tbl_v, dtm_v,
               iu_v, iv_v, res_v):
    c = lax.axis_index("c")
    s = lax.axis_index("s")
    wid = s * 2 + c
    base = wid * _EPW
    pltpu.sync_copy(ptsT_hbm, tbl_v)
    pltpu.sync_copy(dtm_hbm, dtm_v)
    pltpu.sync_copy(eu_hbm.at[pl.ds(base, _EPW)], iu_v)
    pltpu.sync_copy(ev_hbm.at[pl.ds(base, _EPW)], iv_v)
    for i in range(_EPW // 16):
        u = iu_v[pl.ds(i * 16, 16)]
        v = iv_v[pl.ds(i * 16, 16)]
        xu = plsc.load_gather(tbl_v, [u])
        xv = plsc.load_gather(tbl_v, [v])
        yu = plsc.load_gather(tbl_v, [u + _N])
        yv = plsc.load_gather(tbl_v, [v + _N])
        zu = plsc.load_gather(tbl_v, [u + 2 * _N])
        zv = plsc.load_gather(tbl_v, [v + 2 * _N])
        fu = plsc.load_gather(dtm_v, [u])
        fv = plsc.load_gather(dtm_v, [v])
        dx = xu - xv
        dy = yu - yv
        dz = zu - zv
        s2 = dx * dx + dy * dy + dz * dz + 1e-12
        ib = plsc.bitcast(s2, jnp.int32)
        yb = lax.shift_right_logical(ib, jnp.int32(1)) + jnp.int32(0x1FBD1DF5)
        y = plsc.bitcast(yb, jnp.float32)
        y = 0.5 * (y + s2 / y)
        y = 0.5 * (y + s2 / y)
        y = 0.5 * (y + s2 / y)
        res_v[pl.ds(i * 16, 16)] = y + jnp.maximum(fu, fv)
    pltpu.sync_copy(res_v, out_hbm.at[pl.ds(base, _EPW)])


def _edge_call(ptsT_flat, dtm, eu, ev):
    mesh = plsc.VectorSubcoreMesh(core_axis_name="c", subcore_axis_name="s")
    run = pl.kernel(
        _edge_body,
        out_type=jax.ShapeDtypeStruct((_EPAD,), jnp.float32),
        mesh=mesh,
        compiler_params=pltpu.CompilerParams(needs_layout_passes=False),
        scratch_types=[
            pltpu.VMEM((3 * _N,), jnp.float32),
            pltpu.VMEM((_N,), jnp.float32),
            pltpu.VMEM((_EPW,), jnp.int32),
            pltpu.VMEM((_EPW,), jnp.int32),
            pltpu.VMEM((_EPW,), jnp.float32),
        ],
    )
    return run(ptsT_flat, dtm, eu, ev)


def kernel(pts, edges):
    ptsT = pts.T
    dtm = _dtm(pts, ptsT)
    eu = jnp.pad(edges[:, 0], (0, _EPAD - _E))
    ev = jnp.pad(edges[:, 1], (0, _EPAD - _E))
    out = _edge_call(ptsT.reshape(-1), dtm, eu, ev)
    return out[:_E]


# speedup vs baseline: 1.0263x; 1.0263x over previous
import jax
import jax.numpy as jnp
from jax import lax
from jax.experimental import pallas as pl
from jax.experimental.pallas import tpu as pltpu
from jax.experimental.pallas import tpu_sc as plsc

_K = 10
_N = 8192
_E = 50000


_R = 1024


def _oem_sort_network(n):
    comps = []

    def oddeven_merge(lo, n2, r):
        step = r * 2
        if step < n2:
            oddeven_merge(lo, n2, step)
            oddeven_merge(lo + r, n2, step)
            for i in range(lo + r, lo + n2 - r, step):
                comps.append((i, i + r))
        else:
            comps.append((lo, lo + r))

    def sort_range(lo, hi):
        if hi - lo >= 1:
            mid = lo + (hi - lo) // 2
            sort_range(lo, mid)
            sort_range(mid + 1, hi)
            oddeven_merge(lo, hi - lo + 1, 1)

    sort_range(0, n - 1)
    return comps


def _pruned_topk_network(n, k):
    needed = set(range(k))
    ops = []
    for (i, j) in reversed(_oem_sort_network(n)):
        ni, nj = i in needed, j in needed
        if not (ni or nj):
            continue
        ops.append((i, j, ni, nj))
        needed.add(i)
        needed.add(j)
    return ops[::-1]


_NET = _pruned_topk_network(_N // 128, _K + 1)


def _dtm_body(pts_blk_ref, ptsT_ref, out_ref):
    bt = ptsT_ref[...]
    a = pts_blk_ref[...]
    sqr = jnp.sum(a * a, axis=1, keepdims=True)
    sqc = jnp.sum(bt * bt, axis=0, keepdims=True)
    G = jnp.dot(
        a.astype(jnp.bfloat16),
        bt.astype(jnp.bfloat16),
        preferred_element_type=jnp.float32,
    )
    Gp = sqc - 2.0 * G
    R = a.shape[0]
    INF = jnp.float32(jnp.inf)

    def insert(lst, x):
        mx = [jnp.maximum(lst[i], x) for i in range(len(lst) - 1)]
        lst[0] = jnp.minimum(lst[0], x)
        for i in range(1, len(lst)):
            lst[i] = jnp.minimum(lst[i], mx[i - 1])

    s = [jnp.full((R, 128), INF, jnp.float32) for _ in range(_K + 1)]
    h = [jnp.full((R, 128), INF, jnp.float32) for _ in range(5)]
    q = [jnp.full((R, 128), INF, jnp.float32) for _ in range(3)]
    r = [jnp.full((R, 128), INF, jnp.float32) for _ in range(2)]
    for v in range(0, _N // 128, 4):
        d = [sqr + Gp[:, (v + u) * 128:(v + u + 1) * 128] for u in range(4)]
        for (i, j) in ((0, 1), (2, 3), (0, 2), (1, 3), (1, 2)):
            lo, hi = jnp.minimum(d[i], d[j]), jnp.maximum(d[i], d[j])
            d[i], d[j] = lo, hi
        insert(s, d[0])
        insert(h, d[1])
        insert(q, d[2])
        insert(r, d[3])
    for x in h + q + r:
        insert(s, x)
    lane = lax.broadcasted_iota(jnp.int32, (R, 128), 1)
    acc = jnp.zeros((R, 1), jnp.float32)
    for t in range(_K + 1):
        m = jnp.min(s[0], axis=1, keepdims=True)
        if t > 0:
            acc = acc + jnp.maximum(m, 1e-12)
        if t < _K:
            hit = s[0] == m
            li = jnp.min(
                jnp.where(hit, lane, jnp.int32(999)), axis=1, keepdims=True
            )
            hf = lane == li
            for i in range(_K):
                s[i] = jnp.where(hf, s[i + 1], s[i])
            s[_K] = jnp.where(hf, INF, s[_K])
    out_ref[...] = jnp.sqrt(acc * (1.0 / _K))


def _dtm(pts, ptsT):
    out = pl.pallas_call(
        _dtm_body,
        grid=(_N // _R,),
        in_specs=[
            pl.BlockSpec((_R, 3), lambda i: (i, 0)),
            pl.BlockSpec((3, _N), lambda i: (0, 0)),
        ],
        out_specs=pl.BlockSpec((_R, 1), lambda i: (i, 0)),
        out_shape=jax.ShapeDtypeStruct((_N, 1), jnp.float32),
    )(pts, ptsT)
    return out[:, 0]


_NW = 32
_EPW = 1568
_EPAD = _NW * _EPW


def _edge_body(ptsT_hbm, dtm_hbm, eu_hbm, ev_hbm, out_hbm, tbl_v, dtm_v,
               iu_v, iv_v, res_v):
    c = lax.axis_index("c")
    s = lax.axis_index("s")
    wid = s * 2 + c
    base = wid * _EPW
    pltpu.sync_copy(ptsT_hbm, tbl_v)
    pltpu.sync_copy(dtm_hbm, dtm_v)
    pltpu.sync_copy(eu_hbm.at[pl.ds(base, _EPW)], iu_v)
    pltpu.sync_copy(ev_hbm.at[pl.ds(base, _EPW)], iv_v)
    for i in range(_EPW // 16):
        u = iu_v[pl.ds(i * 16, 16)]
        v = iv_v[pl.ds(i * 16, 16)]
        xu = plsc.load_gather(tbl_v, [u])
        xv = plsc.load_gather(tbl_v, [v])
        yu = plsc.load_gather(tbl_v, [u + _N])
        yv = plsc.load_gather(tbl_v, [v + _N])
        zu = plsc.load_gather(tbl_v, [u + 2 * _N])
        zv = plsc.load_gather(tbl_v, [v + 2 * _N])
        fu = plsc.load_gather(dtm_v, [u])
        fv = plsc.load_gather(dtm_v, [v])
        dx = xu - xv
        dy = yu - yv
        dz = zu - zv
        s2 = dx * dx + dy * dy + dz * dz + 1e-12
        ib = plsc.bitcast(s2, jnp.int32)
        yb = lax.shift_right_logical(ib, jnp.int32(1)) + jnp.int32(0x1FBD1DF5)
        y = plsc.bitcast(yb, jnp.float32)
        y = 0.5 * (y + s2 / y)
        y = 0.5 * (y + s2 / y)
        y = 0.5 * (y + s2 / y)
        res_v[pl.ds(i * 16, 16)] = y + jnp.maximum(fu, fv)
    pltpu.sync_copy(res_v, out_hbm.at[pl.ds(base, _EPW)])


def _edge_call(ptsT_flat, dtm, eu, ev):
    mesh = plsc.VectorSubcoreMesh(core_axis_name="c", subcore_axis_name="s")
    run = pl.kernel(
        _edge_body,
        out_type=jax.ShapeDtypeStruct((_EPAD,), jnp.float32),
        mesh=mesh,
        compiler_params=pltpu.CompilerParams(needs_layout_passes=False),
        scratch_types=[
            pltpu.VMEM((3 * _N,), jnp.float32),
            pltpu.VMEM((_N,), jnp.float32),
            pltpu.VMEM((_EPW,), jnp.int32),
            pltpu.VMEM((_EPW,), jnp.int32),
            pltpu.VMEM((_EPW,), jnp.float32),
        ],
    )
    return run(ptsT_flat, dtm, eu, ev)


def kernel(pts, edges):
    ptsT = pts.T
    dtm = _dtm(pts, ptsT)
    eu = jnp.pad(edges[:, 0], (0, _EPAD - _E))
    ev = jnp.pad(edges[:, 1], (0, _EPAD - _E))
    out = _edge_call(ptsT.reshape(-1), dtm, eu, ev)
    return out[:_E]


# speedup vs baseline: 1.0600x; 1.0328x over previous
import jax
import jax.numpy as jnp
from jax import lax
from jax.experimental import pallas as pl
from jax.experimental.pallas import tpu as pltpu
from jax.experimental.pallas import tpu_sc as plsc

_K = 10
_N = 8192
_E = 50000


_R = 1024


def _oem_sort_network(n):
    comps = []

    def oddeven_merge(lo, n2, r):
        step = r * 2
        if step < n2:
            oddeven_merge(lo, n2, step)
            oddeven_merge(lo + r, n2, step)
            for i in range(lo + r, lo + n2 - r, step):
                comps.append((i, i + r))
        else:
            comps.append((lo, lo + r))

    def sort_range(lo, hi):
        if hi - lo >= 1:
            mid = lo + (hi - lo) // 2
            sort_range(lo, mid)
            sort_range(mid + 1, hi)
            oddeven_merge(lo, hi - lo + 1, 1)

    sort_range(0, n - 1)
    return comps


def _pruned_topk_network(n, k):
    needed = set(range(k))
    ops = []
    for (i, j) in reversed(_oem_sort_network(n)):
        ni, nj = i in needed, j in needed
        if not (ni or nj):
            continue
        ops.append((i, j, ni, nj))
        needed.add(i)
        needed.add(j)
    return ops[::-1]


_NET = _pruned_topk_network(_N // 128, _K + 1)


def _dtm_body(pts_blk_ref, ptsT_ref, out_ref):
    bt = ptsT_ref[...]
    a = pts_blk_ref[...]
    sqr = jnp.sum(a * a, axis=1, keepdims=True)
    sqc = jnp.sum(bt * bt, axis=0, keepdims=True)
    G = jnp.dot(
        a.astype(jnp.bfloat16),
        bt.astype(jnp.bfloat16),
        preferred_element_type=jnp.float32,
    )
    Gp = sqc - 2.0 * G
    R = a.shape[0]
    INF = jnp.float32(jnp.inf)

    def insert(lst, x):
        mx = [jnp.maximum(lst[i], x) for i in range(len(lst) - 1)]
        lst[0] = jnp.minimum(lst[0], x)
        for i in range(1, len(lst)):
            lst[i] = jnp.minimum(lst[i], mx[i - 1])

    s = [jnp.full((R, 128), INF, jnp.float32) for _ in range(_K + 1)]
    h = [jnp.full((R, 128), INF, jnp.float32) for _ in range(5)]
    q = [jnp.full((R, 128), INF, jnp.float32) for _ in range(3)]
    r = [jnp.full((R, 128), INF, jnp.float32) for _ in range(2)]
    for v in range(0, _N // 128, 4):
        d = [sqr + Gp[:, (v + u) * 128:(v + u + 1) * 128] for u in range(4)]
        for (i, j) in ((0, 1), (2, 3), (0, 2), (1, 3), (1, 2)):
            lo, hi = jnp.minimum(d[i], d[j]), jnp.maximum(d[i], d[j])
            d[i], d[j] = lo, hi
        insert(s, d[0])
        insert(h, d[1])
        insert(q, d[2])
        insert(r, d[3])
    for x in h + q + r:
        insert(s, x)
    lane = lax.broadcasted_iota(jnp.int32, (R, 128), 1)
    acc = jnp.zeros((R, 1), jnp.float32)
    for t in range(_K + 1):
        m = jnp.min(s[0], axis=1, keepdims=True)
        if t > 0:
            acc = acc + jnp.maximum(m, 1e-12)
        if t < _K:
            hit = s[0] == m
            li = jnp.min(
                jnp.where(hit, lane, jnp.int32(999)), axis=1, keepdims=True
            )
            hf = lane == li
            for i in range(_K):
                s[i] = jnp.where(hf, s[i + 1], s[i])
            s[_K] = jnp.where(hf, INF, s[_K])
    out_ref[...] = jnp.sqrt(acc * (1.0 / _K))


def _dtm(pts, ptsT):
    out = pl.pallas_call(
        _dtm_body,
        grid=(_N // _R,),
        in_specs=[
            pl.BlockSpec((_R, 3), lambda i: (i, 0)),
            pl.BlockSpec((3, _N), lambda i: (0, 0)),
        ],
        out_specs=pl.BlockSpec((_R, 1), lambda i: (i, 0)),
        out_shape=jax.ShapeDtypeStruct((_N, 1), jnp.float32),
    )(pts, ptsT)
    return out[:, 0]


_NW = 32
_EPW = 1568
_EPAD = _NW * _EPW


def _geo_body(ptsT_hbm, eu_hbm, ev_hbm, out_hbm, tbl_v, iu_v, iv_v, res_v):
    c = lax.axis_index("c")
    s = lax.axis_index("s")
    wid = s * 2 + c
    base = wid * _EPW
    pltpu.sync_copy(ptsT_hbm, tbl_v)
    pltpu.sync_copy(eu_hbm.at[pl.ds(base, _EPW)], iu_v)
    pltpu.sync_copy(ev_hbm.at[pl.ds(base, _EPW)], iv_v)
    for i in range(_EPW // 16):
        u = iu_v[pl.ds(i * 16, 16)]
        v = iv_v[pl.ds(i * 16, 16)]
        xu = plsc.load_gather(tbl_v, [u])
        xv = plsc.load_gather(tbl_v, [v])
        yu = plsc.load_gather(tbl_v, [u + _N])
        yv = plsc.load_gather(tbl_v, [v + _N])
        zu = plsc.load_gather(tbl_v, [u + 2 * _N])
        zv = plsc.load_gather(tbl_v, [v + 2 * _N])
        dx = xu - xv
        dy = yu - yv
        dz = zu - zv
        s2 = dx * dx + dy * dy + dz * dz + 1e-12
        ib = plsc.bitcast(s2, jnp.int32)
        yb = lax.shift_right_logical(ib, jnp.int32(1)) + jnp.int32(0x1FBD1DF5)
        y = plsc.bitcast(yb, jnp.float32)
        y = 0.5 * (y + s2 / y)
        y = 0.5 * (y + s2 / y)
        y = 0.5 * (y + s2 / y)
        res_v[pl.ds(i * 16, 16)] = y
    pltpu.sync_copy(res_v, out_hbm.at[pl.ds(base, _EPW)])


def _fin_body(dtm_hbm, eu_hbm, ev_hbm, d_hbm, out_hbm, dtm_v, iu_v, iv_v,
              d_v, res_v):
    c = lax.axis_index("c")
    s = lax.axis_index("s")
    wid = s * 2 + c
    base = wid * _EPW
    pltpu.sync_copy(dtm_hbm, dtm_v)
    pltpu.sync_copy(eu_hbm.at[pl.ds(base, _EPW)], iu_v)
    pltpu.sync_copy(ev_hbm.at[pl.ds(base, _EPW)], iv_v)
    pltpu.sync_copy(d_hbm.at[pl.ds(base, _EPW)], d_v)
    for i in range(_EPW // 16):
        u = iu_v[pl.ds(i * 16, 16)]
        v = iv_v[pl.ds(i * 16, 16)]
        fu = plsc.load_gather(dtm_v, [u])
        fv = plsc.load_gather(dtm_v, [v])
        res_v[pl.ds(i * 16, 16)] = d_v[pl.ds(i * 16, 16)] + jnp.maximum(fu, fv)
    pltpu.sync_copy(res_v, out_hbm.at[pl.ds(base, _EPW)])


def _edge_calls(ptsT_flat, eu, ev):
    mesh = plsc.VectorSubcoreMesh(core_axis_name="c", subcore_axis_name="s")
    geo = pl.kernel(
        _geo_body,
        out_type=jax.ShapeDtypeStruct((_EPAD,), jnp.float32),
        mesh=mesh,
        compiler_params=pltpu.CompilerParams(needs_layout_passes=False),
        scratch_types=[
            pltpu.VMEM((3 * _N,), jnp.float32),
            pltpu.VMEM((_EPW,), jnp.int32),
            pltpu.VMEM((_EPW,), jnp.int32),
            pltpu.VMEM((_EPW,), jnp.float32),
        ],
    )
    fin = pl.kernel(
        _fin_body,
        out_type=jax.ShapeDtypeStruct((_EPAD,), jnp.float32),
        mesh=mesh,
        compiler_params=pltpu.CompilerParams(needs_layout_passes=False),
        scratch_types=[
            pltpu.VMEM((_N,), jnp.float32),
            pltpu.VMEM((_EPW,), jnp.int32),
            pltpu.VMEM((_EPW,), jnp.int32),
            pltpu.VMEM((_EPW,), jnp.float32),
            pltpu.VMEM((_EPW,), jnp.float32),
        ],
    )
    d = geo(ptsT_flat, eu, ev)
    return fin, d


def kernel(pts, edges):
    ptsT = pts.T
    eu = jnp.pad(edges[:, 0], (0, _EPAD - _E))
    ev = jnp.pad(edges[:, 1], (0, _EPAD - _E))
    fin, d = _edge_calls(ptsT.reshape(-1), eu, ev)
    dtm = _dtm(pts, ptsT)
    out = fin(dtm, eu, ev, d)
    return out[:_E]
